# baseline (speedup 1.0000x reference)
import functools

import jax
import jax.numpy as jnp
from jax import lax
from jax.experimental import pallas as pl
from jax.experimental.pallas import tpu as pltpu
from jax.experimental.pallas import tpu_sc as plsc

_FIELD_IDX = (0,) * 7 + (1,) * 7 + (2,) * 6 + (3,) * 6
_NF = 26
_NT = 4
_D = 16
_B = 4096
_ROWS = 260000
_PAIRS = _NF * (_NF - 1) // 2
_PADF = 32

_NC = 2
_NS = 16
_NW = _NC * _NS
_BPW = _B // _NW
_G = 2
_NG = _BPW // _G
_NBUF = 2
_IPW = _BPW * _PADF


def _fused_body(w_hbm, gidx_hbm, out_hbm, idx_v, e_v, out_v, sem_g, sem_w):
    wid = lax.axis_index("s") * _NC + lax.axis_index("c")
    b0 = wid * _BPW

    pltpu.sync_copy(gidx_hbm.at[pl.ds(wid * _IPW, _IPW)], idx_v)

    def fire_gather(gg, rb):
        isl = idx_v.at[pl.ds(gg * (_G * _PADF), _G * _PADF)]
        for f in range(_NT):
            pltpu.async_copy(w_hbm.at[f].at[isl], e_v[rb].at[f], sem_g[rb])

    def drain_gather(rb):
        for f in range(_NT):
            pltpu.make_async_copy(
                w_hbm.at[0].at[pl.ds(0, _G * _PADF)], e_v[rb].at[f], sem_g[rb]
            ).wait()

    def fire_writeback(gg, rb):
        pltpu.async_copy(out_v[rb], out_hbm.at[pl.ds(b0 + gg * _G, _G)], sem_w[rb])

    def drain_writeback(rb):
        pltpu.make_async_copy(
            out_v[rb], out_hbm.at[pl.ds(0, _G)], sem_w[rb]
        ).wait()

    def compute(rb):
        for k in range(_G):
            p = 0
            for i in range(_NF - 1):
                fi = _FIELD_IDX[i]
                j = i + 1
                while j < _NF:
                    fj = _FIELD_IDX[j]
                    va = e_v[rb][fj, k * _PADF + i]
                    while j < _NF and _FIELD_IDX[j] == fj:
                        out_v[rb][k, p] = va * e_v[rb][fi, k * _PADF + j]
                        p += 1
                        j += 1

    for rb in range(_NBUF):
        fire_gather(rb, rb)

    def body(g):
        for rb in range(_NBUF):
            gg = g + rb
            drain_gather(rb)

            @pl.when(gg >= _NBUF)
            def _():
                drain_writeback(rb)

            compute(rb)
            fire_writeback(gg, rb)

            @pl.when(gg + _NBUF < _NG)
            def _():
                fire_gather(gg + _NBUF, rb)

    pl.loop(0, _NG, step=_NBUF)(body)

    for rb in range(_NBUF):
        drain_writeback(rb)


@functools.cache
def _sc_fused():
    return functools.partial(
        pl.kernel,
        mesh=plsc.VectorSubcoreMesh(core_axis_name="c", subcore_axis_name="s"),
        out_type=jax.ShapeDtypeStruct((_B, _PAIRS, _D), jnp.float32),
        scratch_types=[
            pltpu.VMEM((_IPW,), jnp.int32),
            [pltpu.VMEM((_NT, _G * _PADF, _D), jnp.float32) for _ in range(_NBUF)],
            [pltpu.VMEM((_G, _PAIRS, _D), jnp.float32) for _ in range(_NBUF)],
            [pltpu.SemaphoreType.DMA for _ in range(_NBUF)],
            [pltpu.SemaphoreType.DMA for _ in range(_NBUF)],
        ],
        compiler_params=pltpu.CompilerParams(use_tc_tiling_on_sc=False),
    )(_fused_body)


def kernel(x, W):
    xi = x + (jnp.arange(_NF, dtype=x.dtype) * 10000)[None, :]
    xip = jnp.concatenate(
        [xi, jnp.zeros((_B, _PADF - _NF), dtype=xi.dtype)], axis=1
    )
    gidx = xip.reshape(_B * _PADF)
    return _sc_fused()(W, gidx)


# speedup vs baseline: 2.1328x; 2.1328x over previous
import functools

import jax
import jax.numpy as jnp
from jax import lax
from jax.experimental import pallas as pl
from jax.experimental.pallas import tpu as pltpu
from jax.experimental.pallas import tpu_sc as plsc

_FIELD_IDX = (0,) * 7 + (1,) * 7 + (2,) * 6 + (3,) * 6
_NF = 26
_NT = 4
_D = 16
_B = 4096
_ROWS = 260000
_PAIRS = _NF * (_NF - 1) // 2
_PADF = 32

_NC = 2
_NS = 16
_NW = _NC * _NS
_BPW = _B // _NW
_G = 2
_NG = _BPW // _G
_NBUF = 2
_IPW = _BPW * _PADF


def _fused_body(w_hbm, gidx_hbm, out_hbm, idx_v, e_v, out_v, sem_g, sem_w):
    wid = lax.axis_index("s") * _NC + lax.axis_index("c")
    b0 = wid * _BPW

    pltpu.sync_copy(gidx_hbm.at[pl.ds(wid * _IPW, _IPW)], idx_v)

    def fire_gather(gg, rb):
        isl = idx_v.at[pl.ds(gg * (_G * _PADF), _G * _PADF)]
        for f in range(_NT):
            pltpu.async_copy(w_hbm.at[f].at[isl], e_v[rb].at[f], sem_g[rb])

    def drain_gather(rb):
        for f in range(_NT):
            pltpu.make_async_copy(
                w_hbm.at[0].at[pl.ds(0, _G * _PADF)], e_v[rb].at[f], sem_g[rb]
            ).wait()

    def fire_writeback(gg, rb):
        pltpu.async_copy(out_v[rb], out_hbm.at[pl.ds(b0 + gg * _G, _G)], sem_w[rb])

    def drain_writeback(rb):
        pltpu.make_async_copy(
            out_v[rb], out_hbm.at[pl.ds(0, _G)], sem_w[rb]
        ).wait()

    def compute(rb):
        for k in range(_G):
            p = 0
            for i in range(_NF - 1):
                fi = _FIELD_IDX[i]
                j = i + 1
                while j < _NF:
                    fj = _FIELD_IDX[j]
                    va = e_v[rb][fj, k * _PADF + i]
                    while j < _NF and _FIELD_IDX[j] == fj:
                        out_v[rb][k, pl.ds(p * _D, _D)] = (
                            va * e_v[rb][fi, k * _PADF + j]
                        )
                        p += 1
                        j += 1

    for rb in range(_NBUF):
        fire_gather(rb, rb)

    def body(g):
        for rb in range(_NBUF):
            gg = g + rb
            drain_gather(rb)

            @pl.when(gg >= _NBUF)
            def _():
                drain_writeback(rb)

            compute(rb)
            fire_writeback(gg, rb)

            @pl.when(gg + _NBUF < _NG)
            def _():
                fire_gather(gg + _NBUF, rb)

    pl.loop(0, _NG, step=_NBUF)(body)

    for rb in range(_NBUF):
        drain_writeback(rb)


@functools.cache
def _sc_fused():
    return functools.partial(
        pl.kernel,
        mesh=plsc.VectorSubcoreMesh(core_axis_name="c", subcore_axis_name="s"),
        out_type=jax.ShapeDtypeStruct((_B, _PAIRS * _D), jnp.float32),
        scratch_types=[
            pltpu.VMEM((_IPW,), jnp.int32),
            [pltpu.VMEM((_NT, _G * _PADF, _D), jnp.float32) for _ in range(_NBUF)],
            [pltpu.VMEM((_G, _PAIRS * _D), jnp.float32) for _ in range(_NBUF)],
            [pltpu.SemaphoreType.DMA for _ in range(_NBUF)],
            [pltpu.SemaphoreType.DMA for _ in range(_NBUF)],
        ],
        compiler_params=pltpu.CompilerParams(use_tc_tiling_on_sc=False),
    )(_fused_body)


def kernel(x, W):
    xi = x + (jnp.arange(_NF, dtype=x.dtype) * 10000)[None, :]
    xip = jnp.concatenate(
        [xi, jnp.zeros((_B, _PADF - _NF), dtype=xi.dtype)], axis=1
    )
    gidx = xip.reshape(_B * _PADF)
    out = _sc_fused()(W, gidx)
    return out.reshape(_B, _PAIRS, _D)
